# baseline (speedup 1.0000x reference)
import functools

import jax
import jax.numpy as jnp
from jax import lax
from jax.experimental import pallas as pl
from jax.experimental.pallas import tpu as pltpu
from jax.experimental.pallas import tpu_sc as plsc

VOCAB = 1000000
SEQ = 200
BATCH = 4096
DIM = 64

NC, NS = 2, 16
NW = NC * NS
SB = SEQ // 8
BT = BATCH // 128
NITEM = SB * BT // NW
LANES = 16

_mesh = plsc.VectorSubcoreMesh(core_axis_name="c", subcore_axis_name="s")


@functools.partial(
    pl.kernel,
    mesh=_mesh,
    compiler_params=pltpu.CompilerParams(
        use_tc_tiling_on_sc=False, needs_layout_passes=False
    ),
    out_type=jax.ShapeDtypeStruct((SEQ, 8, BT, 8, 128), jnp.float32),
    scratch_types=[
        pltpu.VMEM((SEQ, DIM), jnp.float32),
        pltpu.VMEM((2, 8, 128), jnp.int32),
        pltpu.VMEM((8 * 128, DIM), jnp.float32),
        pltpu.VMEM((8, 8, 128), jnp.float32),
        pltpu.SemaphoreType.DMA,
        pltpu.SemaphoreType.DMA,
        pltpu.SemaphoreType.DMA,
        pltpu.SemaphoreType.DMA,
    ],
)
def _embed(idx_hbm, table_hbm, pos_hbm, out_hbm,
           pos_v, idx_v, rows_v, st_v, sem0, sem1, sem2, sem3):
    sems = (sem0, sem1, sem2, sem3)
    wid = lax.axis_index("s") * NC + lax.axis_index("c")
    item0 = wid * NITEM
    pltpu.sync_copy(pos_hbm, pos_v)
    iota16 = lax.iota(jnp.int32, LANES)

    def stage_idx(it, slot):
        sb = it // BT
        bt = lax.rem(it, BT)
        pltpu.sync_copy(
            idx_hbm.at[pl.ds(sb * 8, 8), pl.ds(bt * 128, 128)],
            idx_v.at[slot],
        )

    def issue_gather(islot, si):
        return pltpu.async_copy(
            table_hbm.at[idx_v.at[islot, si]],
            rows_v.at[pl.ds(si * 128, 128)],
            sems[si % 4],
        )

    stage_idx(item0, 0)
    for si in range(4):
        issue_gather(0, si)

    def item_body(i, carry):
        it = item0 + i
        sb = it // BT
        bt = lax.rem(it, BT)
        islot = lax.rem(i, 2)

        @pl.when(i + 1 < NITEM)
        def _stage_next():
            stage_idx(it + 1, lax.rem(i + 1, 2))

        for si in range(8):
            pltpu.make_async_copy(
                table_hbm.at[idx_v.at[islot, si]],
                rows_v.at[pl.ds(si * 128, 128)],
                sems[si % 4],
            ).wait()

            s = sb * 8 + si
            rvecs = [iota16 + (si * 128 + bg * LANES) for bg in range(8)]

            def dt_body(dt, c2, _si=si, _s=s, _rvecs=rvecs):
                svec = jnp.full((LANES,), _s, jnp.int32)
                for dp in range(8):
                    c = dt * 8 + dp
                    cvec = jnp.full((LANES,), c, jnp.int32)
                    pv = plsc.load_gather(pos_v, [svec, cvec])
                    for bg in range(8):
                        vals = plsc.load_gather(rows_v, [_rvecs[bg], cvec])
                        st_v[dt, dp, pl.ds(bg * LANES, LANES)] = vals + pv
                return c2

            lax.fori_loop(0, 8, dt_body, 0)

            if si < 4:
                issue_gather(islot, si + 4)
            else:
                @pl.when(i + 1 < NITEM)
                def _issue_next(_si=si):
                    issue_gather(lax.rem(i + 1, 2), _si - 4)

            pltpu.sync_copy(st_v, out_hbm.at[s, :, bt])
        return carry

    lax.fori_loop(0, NITEM, item_body, 0)


def kernel(inputs, token_table, pos_table):
    idx_t = inputs.T
    out5 = _embed(idx_t, token_table, pos_table)
    return out5.transpose(2, 4, 0, 1, 3).reshape(BATCH, SEQ, DIM)


# speedup vs baseline: 1.7392x; 1.7392x over previous
import functools

import jax
import jax.numpy as jnp
from jax import lax
from jax.experimental import pallas as pl
from jax.experimental.pallas import tpu as pltpu
from jax.experimental.pallas import tpu_sc as plsc

VOCAB = 1000000
SEQ = 200
BATCH = 4096
DIM = 64

NC, NS = 2, 16
NW = NC * NS
SB = SEQ // 8
BT = BATCH // 128
NITEM = SB * BT // NW
LANES = 16
STP = 129

_mesh = plsc.VectorSubcoreMesh(core_axis_name="c", subcore_axis_name="s")


@functools.partial(
    pl.kernel,
    mesh=_mesh,
    compiler_params=pltpu.CompilerParams(
        use_tc_tiling_on_sc=False, needs_layout_passes=False
    ),
    out_type=jax.ShapeDtypeStruct((SEQ, 8, BT, 8, 128), jnp.float32),
    scratch_types=[
        pltpu.VMEM((SEQ, DIM), jnp.float32),
        pltpu.VMEM((3, 8, 128), jnp.int32),
        pltpu.VMEM((8 * 128, DIM), jnp.float32),
        pltpu.VMEM((8, 8, STP), jnp.float32),
        pltpu.SemaphoreType.DMA,
        pltpu.SemaphoreType.DMA,
        pltpu.SemaphoreType.DMA,
        pltpu.SemaphoreType.DMA,
        pltpu.SemaphoreType.DMA,
        pltpu.SemaphoreType.DMA,
        pltpu.SemaphoreType.DMA,
        pltpu.SemaphoreType.DMA,
    ],
)
def _embed(idx_hbm, table_hbm, pos_hbm, out_hbm,
           pos_v, idx_v, rows_v, st_v, *sems):
    wid = lax.axis_index("s") * NC + lax.axis_index("c")
    item0 = wid * NITEM
    pltpu.sync_copy(pos_hbm, pos_v)
    iota16 = lax.iota(jnp.int32, LANES)
    dtv = [(iota16 + cg * LANES) // 8 for cg in range(4)]
    dpv = [lax.rem(iota16 + cg * LANES, 8) for cg in range(4)]

    def stage_idx(it, slot):
        sb = it // BT
        bt = lax.rem(it, BT)
        pltpu.sync_copy(
            idx_hbm.at[pl.ds(sb * 8, 8), pl.ds(bt * 128, 128)],
            idx_v.at[slot],
        )

    def issue_gather(islot, si):
        return pltpu.async_copy(
            table_hbm.at[idx_v.at[islot, si]],
            rows_v.at[pl.ds(si * 128, 128)],
            sems[si],
        )

    stage_idx(item0, 0)
    for si in range(8):
        issue_gather(0, si)

    def item_body(i, carry):
        it = item0 + i
        sb = it // BT
        bt = lax.rem(it, BT)
        islot = lax.rem(i, 3)
        nslot = lax.rem(i + 1, 3)

        @pl.when(i + 1 < NITEM)
        def _stage_next():
            stage_idx(it + 1, nslot)

        for si in range(8):
            pltpu.make_async_copy(
                table_hbm.at[idx_v.at[islot, si]],
                rows_v.at[pl.ds(si * 128, 128)],
                sems[si],
            ).wait()

            s = sb * 8 + si
            pvecs = [pos_v[s, pl.ds(cg * LANES, LANES)] for cg in range(4)]

            def b_body(t, c2, _si=si, _pv=pvecs):
                for u in range(4):
                    b = t * 4 + u
                    bv = jnp.full((LANES,), b, jnp.int32)
                    for cg in range(4):
                        vals = rows_v[_si * 128 + b, pl.ds(cg * LANES, LANES)]
                        plsc.store_scatter(
                            st_v, [dtv[cg], dpv[cg], bv], vals + _pv[cg]
                        )
                return c2

            lax.fori_loop(0, 32, b_body, 0)

            if si == 7:
                pass
            @pl.when(i + 1 < NITEM)
            def _refill(_si=si):
                issue_gather(nslot, _si)

            pltpu.sync_copy(
                st_v.at[:, :, pl.ds(0, 128)], out_hbm.at[s, :, bt]
            )
        return carry

    lax.fori_loop(0, NITEM, item_body, 0)


def kernel(inputs, token_table, pos_table):
    idx_t = inputs.T
    out5 = _embed(idx_t, token_table, pos_table)
    return out5.transpose(2, 4, 0, 1, 3).reshape(BATCH, SEQ, DIM)


# speedup vs baseline: 1.7477x; 1.0049x over previous
import functools

import jax
import jax.numpy as jnp
from jax import lax
from jax.experimental import pallas as pl
from jax.experimental.pallas import tpu as pltpu
from jax.experimental.pallas import tpu_sc as plsc

VOCAB = 1000000
SEQ = 200
BATCH = 4096
DIM = 64

NC, NS = 2, 16
NW = NC * NS
SB = SEQ // 8
BT = BATCH // 128
NITEM = SB * BT // NW
LANES = 16
STP = 129

_mesh = plsc.VectorSubcoreMesh(core_axis_name="c", subcore_axis_name="s")


@functools.partial(
    pl.kernel,
    mesh=_mesh,
    compiler_params=pltpu.CompilerParams(
        use_tc_tiling_on_sc=False, needs_layout_passes=False
    ),
    out_type=jax.ShapeDtypeStruct((SEQ, 8, BT, 8, 128), jnp.float32),
    scratch_types=[
        pltpu.VMEM((SEQ, DIM), jnp.float32),
        pltpu.VMEM((3, 8, 128), jnp.int32),
        pltpu.VMEM((8 * 128, DIM), jnp.float32),
        pltpu.VMEM((8, 8, STP), jnp.float32),
        pltpu.SemaphoreType.DMA,
        pltpu.SemaphoreType.DMA,
        pltpu.SemaphoreType.DMA,
        pltpu.SemaphoreType.DMA,
        pltpu.SemaphoreType.DMA,
        pltpu.SemaphoreType.DMA,
        pltpu.SemaphoreType.DMA,
        pltpu.SemaphoreType.DMA,
    ],
)
def _embed(idx_hbm, table_hbm, pos_hbm, out_hbm,
           pos_v, idx_v, rows_v, st_v, *sems):
    wid = lax.axis_index("s") * NC + lax.axis_index("c")
    item0 = wid * NITEM
    pltpu.sync_copy(pos_hbm, pos_v)
    iota16 = lax.iota(jnp.int32, LANES)
    dtv = [(iota16 + cg * LANES) // 8 for cg in range(4)]
    dpv = [lax.rem(iota16 + cg * LANES, 8) for cg in range(4)]

    def stage_idx(it, slot):
        sb = it // BT
        bt = lax.rem(it, BT)
        pltpu.sync_copy(idx_hbm.at[sb, bt], idx_v.at[slot])

    def issue_gather(islot, si):
        return pltpu.async_copy(
            table_hbm.at[idx_v.at[islot, si]],
            rows_v.at[pl.ds(si * 128, 128)],
            sems[si],
        )

    stage_idx(item0, 0)
    for si in range(8):
        issue_gather(0, si)

    def item_body(i, carry):
        it = item0 + i
        sb = it // BT
        bt = lax.rem(it, BT)
        islot = lax.rem(i, 3)
        nslot = lax.rem(i + 1, 3)

        @pl.when(i + 1 < NITEM)
        def _stage_next():
            stage_idx(it + 1, nslot)

        for si in range(8):
            pltpu.make_async_copy(
                table_hbm.at[idx_v.at[islot, si]],
                rows_v.at[pl.ds(si * 128, 128)],
                sems[si],
            ).wait()

            s = sb * 8 + si
            pvecs = [pos_v[s, pl.ds(cg * LANES, LANES)] for cg in range(4)]

            def b_body(t, c2, _si=si, _pv=pvecs):
                for u in range(4):
                    b = t * 4 + u
                    bv = jnp.full((LANES,), b, jnp.int32)
                    for cg in range(4):
                        vals = rows_v[_si * 128 + b, pl.ds(cg * LANES, LANES)]
                        plsc.store_scatter(
                            st_v, [dtv[cg], dpv[cg], bv], vals + _pv[cg]
                        )
                return c2

            lax.fori_loop(0, 32, b_body, 0)

            if si == 7:
                pass
            @pl.when(i + 1 < NITEM)
            def _refill(_si=si):
                issue_gather(nslot, _si)

            pltpu.sync_copy(
                st_v.at[:, :, pl.ds(0, 128)], out_hbm.at[s, :, bt]
            )
        return carry

    lax.fori_loop(0, NITEM, item_body, 0)


def kernel(inputs, token_table, pos_table):
    idx4 = inputs.reshape(BT, 128, SB, 8).transpose(2, 0, 3, 1)
    out5 = _embed(idx4, token_table, pos_table)
    return out5.transpose(2, 4, 0, 1, 3).reshape(BATCH, SEQ, DIM)
